# baseline (speedup 1.0000x reference)
import functools

import jax
import jax.numpy as jnp
from jax import lax
from jax.experimental import pallas as pl
from jax.experimental.pallas import tpu as pltpu
from jax.experimental.pallas import tpu_sc as plsc

_STEPS = 128
_X_MIN = 0.0
_X_MAX = 1.0
_L = 16
_R = 416
_NBUF = 2


def _make_sc_kernel(n_rows: int):
    info = plsc.get_sparse_core_info()
    nc, ns = info.num_cores, info.num_subcores
    nw = nc * ns
    assert n_rows % (nw * _R) == 0
    rows_per_w = n_rows // nw
    chunks_per_w = rows_per_w // _R
    chunk_elems = _R * _STEPS

    mesh = plsc.VectorSubcoreMesh(core_axis_name="c", subcore_axis_name="s")

    @functools.partial(
        pl.kernel,
        mesh=mesh,
        out_type=jax.ShapeDtypeStruct((n_rows * _STEPS,), jnp.float32),
        compiler_params=pltpu.CompilerParams(needs_layout_passes=False),
        scratch_types=[
            pltpu.VMEM((rows_per_w,), jnp.float32),
            pltpu.VMEM((chunk_elems,), jnp.float32),
            pltpu.VMEM((chunk_elems,), jnp.float32),
            pltpu.VMEM((_R,), jnp.int32),
            pltpu.VMEM((_R,), jnp.int32),
            pltpu.SemaphoreType.DMA,
            pltpu.SemaphoreType.DMA,
        ],
    )
    def k(x_hbm, out_hbm, x_v, buf0, buf1, pos0, pos1, sem0, sem1):
        wid = lax.axis_index("s") * nc + lax.axis_index("c")
        row_base = wid * rows_per_w

        pltpu.sync_copy(x_hbm.at[pl.ds(row_base, rows_per_w)], x_v)

        lane = lax.iota(jnp.int32, _L)
        lane_off = lane * _STEPS
        ones = jnp.full((_L,), 1.0, jnp.float32)
        zeros = jnp.zeros((_L,), jnp.float32)

        def _zero(i, _):
            buf0[pl.ds(i * _L, _L)] = zeros
            buf1[pl.ds(i * _L, _L)] = zeros
            return 0
        lax.fori_loop(0, chunk_elems // _L, _zero, 0)

        bufs = (buf0, buf1)
        poss = (pos0, pos1)
        sems = (sem0, sem1)

        def _chunk(c, buf, posv, sem):
            rel = c * _R

            @pl.when(c >= _NBUF)
            def _():
                pltpu.make_async_copy(
                    buf, out_hbm.at[pl.ds((row_base + rel) * _STEPS,
                                          chunk_elems)], sem).wait()
                for g in range(_R // _L):
                    pv = posv[pl.ds(g * _L, _L)]
                    plsc.store_scatter(buf, [pv], zeros)

            for g in range(_R // _L):
                xv = x_v[pl.ds(rel + g * _L, _L)]
                xs = (xv - _X_MIN) * ((_STEPS - 1) / (_X_MAX - _X_MIN))
                idx = jnp.clip(xs, 0.0, float(_STEPS - 1)).astype(jnp.int32)
                pos = idx + (g * _L) * _STEPS + lane_off
                posv[pl.ds(g * _L, _L)] = pos
                plsc.store_scatter(buf, [pos], ones)

            pltpu.make_async_copy(
                buf, out_hbm.at[pl.ds((row_base + rel) * _STEPS,
                                      chunk_elems)], sem).start()

        def _step(t, _):
            for b in range(_NBUF):
                _chunk(t * _NBUF + b, bufs[b], poss[b], sems[b])
            return 0
        lax.fori_loop(0, chunks_per_w // _NBUF, _step, 0)

        for b in range(_NBUF):
            pltpu.make_async_copy(
                bufs[b], out_hbm.at[pl.ds(row_base * _STEPS, chunk_elems)],
                sems[b]).wait()

    return k


def kernel(x, eye):
    del eye
    n, c = x.shape
    n_rows = n * c
    out_flat = _make_sc_kernel(n_rows)(x.reshape(n_rows))
    return out_flat.reshape(n, c, _STEPS)


# speedup vs baseline: 1.9091x; 1.9091x over previous
import functools

import jax
import jax.numpy as jnp
from jax import lax
from jax.experimental import pallas as pl
from jax.experimental.pallas import tpu as pltpu
from jax.experimental.pallas import tpu_sc as plsc

_STEPS = 128
_X_MIN = 0.0
_X_MAX = 1.0
_L = 16
_NSLOT = 8
_SROWS = 32


def _make_sc_kernel(n: int, c: int):
    info = plsc.get_sparse_core_info()
    nc, ns = info.num_cores, info.num_subcores
    nw = nc * ns
    assert n % (nw * _NSLOT) == 0 and c <= _SROWS
    n_per_w = n // nw
    xlen = n_per_w * c
    xlen_pad = (xlen + 2 * _L - 1) // _L * _L

    mesh = plsc.VectorSubcoreMesh(core_axis_name="c", subcore_axis_name="s")

    @functools.partial(
        pl.kernel,
        mesh=mesh,
        out_type=jax.ShapeDtypeStruct((n, c, _STEPS), jnp.float32),
        compiler_params=pltpu.CompilerParams(needs_layout_passes=False),
        scratch_types=(
            [pltpu.VMEM((xlen_pad,), jnp.float32)]
            + [pltpu.VMEM((_SROWS, _STEPS), jnp.float32)] * _NSLOT
            + [pltpu.VMEM((2 * _L,), jnp.int32)] * _NSLOT
            + [pltpu.SemaphoreType.DMA] * _NSLOT
        ),
    )
    def k(x_hbm, out_hbm, *refs):
        x_v = refs[0]
        bufs = refs[1:1 + _NSLOT]
        poss = refs[1 + _NSLOT:1 + 2 * _NSLOT]
        sems = refs[1 + 2 * _NSLOT:1 + 3 * _NSLOT]

        wid = lax.axis_index("s") * nc + lax.axis_index("c")
        i_base = wid * n_per_w

        pltpu.sync_copy(x_hbm.at[pl.ds(i_base * c, xlen)],
                        x_v.at[pl.ds(0, xlen)])

        lane = lax.iota(jnp.int32, _L)
        ones = jnp.full((_L,), 1.0, jnp.float32)
        zeros = jnp.zeros((_L,), jnp.float32)
        jv1 = lane
        jv2 = lane + _L
        m2 = jv2 < c

        def _zero(t, _):
            jj = t // (_STEPS // _L)
            kk = (t % (_STEPS // _L)) * _L
            jv = jnp.full((_L,), jj, jnp.int32)
            kv = kk + lane
            for s in range(_NSLOT):
                plsc.store_scatter(bufs[s], [jv, kv], zeros)
            return 0
        lax.fori_loop(0, c * (_STEPS // _L), _zero, 0)

        def _row(i_loc, buf, posv, sem):
            @pl.when(i_loc >= _NSLOT)
            def _():
                pltpu.make_async_copy(
                    buf.at[pl.ds(0, c)], out_hbm.at[i_base + i_loc],
                    sem).wait()
                k1 = posv[pl.ds(0, _L)]
                k2 = posv[pl.ds(_L, _L)]
                plsc.store_scatter(buf, [jv1, k1], zeros)
                plsc.store_scatter(buf, [jv2, k2], zeros, mask=m2)

            base = i_loc * c
            x1 = x_v[pl.ds(base, _L)]
            x2 = x_v[pl.ds(base + _L, _L)]

            def _q(xv):
                xs = (xv - _X_MIN) * ((_STEPS - 1) / (_X_MAX - _X_MIN))
                return jnp.clip(xs, 0.0, float(_STEPS - 1)).astype(jnp.int32)

            k1 = _q(x1)
            k2 = _q(x2)
            posv[pl.ds(0, _L)] = k1
            posv[pl.ds(_L, _L)] = k2
            plsc.store_scatter(buf, [jv1, k1], ones)
            plsc.store_scatter(buf, [jv2, k2], ones, mask=m2)

            pltpu.make_async_copy(
                buf.at[pl.ds(0, c)], out_hbm.at[i_base + i_loc], sem).start()

        def _step(t, _):
            for s in range(_NSLOT):
                _row(t * _NSLOT + s, bufs[s], poss[s], sems[s])
            return 0
        lax.fori_loop(0, n_per_w // _NSLOT, _step, 0)

        for s in range(_NSLOT):
            pltpu.make_async_copy(
                bufs[s].at[pl.ds(0, c)], out_hbm.at[i_base], sems[s]).wait()

    return k


def kernel(x, eye):
    del eye
    n, c = x.shape
    return _make_sc_kernel(n, c)(x.reshape(n * c))


# speedup vs baseline: 4.9345x; 2.5847x over previous
import functools

import jax
import jax.numpy as jnp
from jax import lax
from jax.experimental import pallas as pl
from jax.experimental.pallas import tpu as pltpu
from jax.experimental.pallas import tpu_sc as plsc

_STEPS = 128
_X_MIN = 0.0
_X_MAX = 1.0
_L = 16
_CH = 256
_NSLOT = 2


def _make_sc_kernel(n: int, c: int):
    info = plsc.get_sparse_core_info()
    nc, ns = info.num_cores, info.num_subcores
    nw = nc * ns
    assert n % (nw * _CH) == 0
    n_per_w = n // nw
    ch_per_j = n_per_w // _CH
    n_chunks = c * ch_per_j
    groups = _CH // _L

    mesh = plsc.VectorSubcoreMesh(core_axis_name="c", subcore_axis_name="s")

    @functools.partial(
        pl.kernel,
        mesh=mesh,
        out_type=jax.ShapeDtypeStruct((c, n, _STEPS), jnp.float32),
        compiler_params=pltpu.CompilerParams(needs_layout_passes=False),
        scratch_types=(
            [pltpu.VMEM((c * n_per_w,), jnp.float32)]
            + [pltpu.VMEM((_CH, _STEPS), jnp.float32)] * _NSLOT
            + [pltpu.VMEM((_CH,), jnp.int32)] * _NSLOT
            + [pltpu.SemaphoreType.DMA] * _NSLOT
        ),
    )
    def k(xt_hbm, out_hbm, *refs):
        x_v = refs[0]
        bufs = refs[1:1 + _NSLOT]
        poss = refs[1 + _NSLOT:1 + 2 * _NSLOT]
        sems = refs[1 + 2 * _NSLOT:1 + 3 * _NSLOT]

        wid = lax.axis_index("s") * nc + lax.axis_index("c")
        i_base = wid * n_per_w

        for j in range(c):
            pltpu.sync_copy(xt_hbm.at[j, pl.ds(i_base, n_per_w)],
                            x_v.at[pl.ds(j * n_per_w, n_per_w)])

        lane = lax.iota(jnp.int32, _L)
        ones = jnp.full((_L,), 1.0, jnp.float32)
        zeros = jnp.zeros((_L,), jnp.float32)
        ivs = [g * _L + lane for g in range(groups)]

        def _zero(t, _):
            rr = t // (_STEPS // _L)
            kk = (t % (_STEPS // _L)) * _L
            rv = jnp.full((_L,), rr, jnp.int32)
            kv = kk + lane
            for s in range(_NSLOT):
                plsc.store_scatter(bufs[s], [rv, kv], zeros)
            return 0
        lax.fori_loop(0, _CH * (_STEPS // _L), _zero, 0)

        def _chunk(t, buf, posv, sem):
            jj = t // ch_per_j
            io = (t % ch_per_j) * _CH

            @pl.when(t >= _NSLOT)
            def _():
                pltpu.make_async_copy(
                    buf, out_hbm.at[jj, pl.ds(i_base + io, _CH)], sem).wait()
                for g in range(groups):
                    pv = posv[pl.ds(g * _L, _L)]
                    plsc.store_scatter(buf, [ivs[g], pv], zeros)

            for g in range(groups):
                xv = x_v[pl.ds(jj * n_per_w + io + g * _L, _L)]
                xs = (xv - _X_MIN) * ((_STEPS - 1) / (_X_MAX - _X_MIN))
                kv = jnp.clip(xs, 0.0, float(_STEPS - 1)).astype(jnp.int32)
                posv[pl.ds(g * _L, _L)] = kv
                plsc.store_scatter(buf, [ivs[g], kv], ones)

            pltpu.make_async_copy(
                buf, out_hbm.at[jj, pl.ds(i_base + io, _CH)], sem).start()

        def _step(t, _):
            for s in range(_NSLOT):
                _chunk(t * _NSLOT + s, bufs[s], poss[s], sems[s])
            return 0
        lax.fori_loop(0, n_chunks // _NSLOT, _step, 0)

        for s in range(_NSLOT):
            pltpu.make_async_copy(
                bufs[s], out_hbm.at[0, pl.ds(i_base, _CH)], sems[s]).wait()

    return k


def kernel(x, eye):
    del eye
    n, c = x.shape
    out_t = _make_sc_kernel(n, c)(x.T)
    return jnp.transpose(out_t, (1, 0, 2))


# speedup vs baseline: 5.6251x; 1.1400x over previous
import functools

import jax
import jax.numpy as jnp
from jax import lax
from jax.experimental import pallas as pl
from jax.experimental.pallas import tpu as pltpu
from jax.experimental.pallas import tpu_sc as plsc

_STEPS = 128
_X_MIN = 0.0
_X_MAX = 1.0
_L = 16
_CH = 256
_NSLOT = 2


def _make_sc_kernel(n: int, c: int):
    info = plsc.get_sparse_core_info()
    nc, ns = info.num_cores, info.num_subcores
    nw = nc * ns
    assert n % (nw * _CH) == 0
    n_per_w = n // nw
    ch_per_j = n_per_w // _CH
    n_chunks = c * ch_per_j
    groups = _CH // _L

    mesh = plsc.VectorSubcoreMesh(core_axis_name="c", subcore_axis_name="s")

    @functools.partial(
        pl.kernel,
        mesh=mesh,
        out_type=jax.ShapeDtypeStruct((c, n, _STEPS), jnp.float32),
        compiler_params=pltpu.CompilerParams(needs_layout_passes=False),
        scratch_types=(
            [pltpu.VMEM((c * n_per_w,), jnp.float32)]
            + [pltpu.VMEM((_CH, _STEPS), jnp.float32)] * _NSLOT
            + [pltpu.VMEM((_CH,), jnp.int32)] * _NSLOT
            + [pltpu.SemaphoreType.DMA] * (_NSLOT + 1)
        ),
    )
    def k(xt_hbm, out_hbm, *refs):
        x_v = refs[0]
        bufs = refs[1:1 + _NSLOT]
        poss = refs[1 + _NSLOT:1 + 2 * _NSLOT]
        sems = refs[1 + 2 * _NSLOT:1 + 3 * _NSLOT]
        xsem = refs[1 + 3 * _NSLOT]

        wid = lax.axis_index("s") * nc + lax.axis_index("c")
        i_base = wid * n_per_w

        xcopies = [
            pltpu.make_async_copy(xt_hbm.at[j, pl.ds(i_base, n_per_w)],
                                  x_v.at[pl.ds(j * n_per_w, n_per_w)], xsem)
            for j in range(c)
        ]
        for cp in xcopies:
            cp.start()

        lane = lax.iota(jnp.int32, _L)
        ones = jnp.full((_L,), 1.0, jnp.float32)
        zeros = jnp.zeros((_L,), jnp.float32)
        ivs = [g * _L + lane for g in range(groups)]

        def _zero(t, _):
            rr = t // (_STEPS // _L)
            kk = (t % (_STEPS // _L)) * _L
            rv = jnp.full((_L,), rr, jnp.int32)
            kv = kk + lane
            for s in range(_NSLOT):
                plsc.store_scatter(bufs[s], [rv, kv], zeros)
            return 0
        lax.fori_loop(0, _CH * (_STEPS // _L), _zero, 0)

        for cp in xcopies:
            cp.wait()

        def _chunk(t, buf, posv, sem):
            jj = t // ch_per_j
            io = (t % ch_per_j) * _CH

            @pl.when(t >= _NSLOT)
            def _():
                pltpu.make_async_copy(
                    buf, out_hbm.at[jj, pl.ds(i_base + io, _CH)], sem).wait()
                for g in range(groups):
                    pv = posv[pl.ds(g * _L, _L)]
                    plsc.store_scatter(buf, [ivs[g], pv], zeros)

            for g in range(groups):
                xv = x_v[pl.ds(jj * n_per_w + io + g * _L, _L)]
                xs = (xv - _X_MIN) * ((_STEPS - 1) / (_X_MAX - _X_MIN))
                kv = jnp.clip(xs, 0.0, float(_STEPS - 1)).astype(jnp.int32)
                posv[pl.ds(g * _L, _L)] = kv
                plsc.store_scatter(buf, [ivs[g], kv], ones)

            pltpu.make_async_copy(
                buf, out_hbm.at[jj, pl.ds(i_base + io, _CH)], sem).start()

        def _step(t, _):
            for s in range(_NSLOT):
                _chunk(t * _NSLOT + s, bufs[s], poss[s], sems[s])
            return 0
        lax.fori_loop(0, n_chunks // _NSLOT, _step, 0)

        for s in range(_NSLOT):
            pltpu.make_async_copy(
                bufs[s], out_hbm.at[0, pl.ds(i_base, _CH)], sems[s]).wait()

    return k


def kernel(x, eye):
    del eye
    n, c = x.shape
    out_t = _make_sc_kernel(n, c)(x.T)
    return jnp.transpose(out_t, (1, 0, 2))


# speedup vs baseline: 5.7838x; 1.0282x over previous
import functools

import jax
import jax.numpy as jnp
from jax import lax
from jax.experimental import pallas as pl
from jax.experimental.pallas import tpu as pltpu
from jax.experimental.pallas import tpu_sc as plsc

_STEPS = 128
_X_MIN = 0.0
_X_MAX = 1.0
_L = 16
_CH = 128
_NSLOT = 4


def _make_sc_kernel(n: int, c: int):
    info = plsc.get_sparse_core_info()
    nc, ns = info.num_cores, info.num_subcores
    nw = nc * ns
    assert n % (nw * _CH) == 0
    n_per_w = n // nw
    ch_per_j = n_per_w // _CH
    n_chunks = c * ch_per_j
    groups = _CH // _L

    mesh = plsc.VectorSubcoreMesh(core_axis_name="c", subcore_axis_name="s")

    @functools.partial(
        pl.kernel,
        mesh=mesh,
        out_type=jax.ShapeDtypeStruct((c, n, _STEPS), jnp.float32),
        compiler_params=pltpu.CompilerParams(needs_layout_passes=False),
        scratch_types=(
            [pltpu.VMEM((c * n_per_w,), jnp.float32)]
            + [pltpu.VMEM((_CH, _STEPS), jnp.float32)] * _NSLOT
            + [pltpu.VMEM((_CH,), jnp.int32)] * _NSLOT
            + [pltpu.SemaphoreType.DMA] * (_NSLOT + 1)
        ),
    )
    def k(xt_hbm, out_hbm, *refs):
        x_v = refs[0]
        bufs = refs[1:1 + _NSLOT]
        poss = refs[1 + _NSLOT:1 + 2 * _NSLOT]
        sems = refs[1 + 2 * _NSLOT:1 + 3 * _NSLOT]
        xsem = refs[1 + 3 * _NSLOT]

        wid = lax.axis_index("s") * nc + lax.axis_index("c")
        i_base = wid * n_per_w

        xcopies = [
            pltpu.make_async_copy(xt_hbm.at[j, pl.ds(i_base, n_per_w)],
                                  x_v.at[pl.ds(j * n_per_w, n_per_w)], xsem)
            for j in range(c)
        ]
        for cp in xcopies:
            cp.start()

        lane = lax.iota(jnp.int32, _L)
        ones = jnp.full((_L,), 1.0, jnp.float32)
        zeros = jnp.zeros((_L,), jnp.float32)
        ivs = [g * _L + lane for g in range(groups)]

        def _zero(t, _):
            rr = t // (_STEPS // _L)
            kk = (t % (_STEPS // _L)) * _L
            rv = jnp.full((_L,), rr, jnp.int32)
            kv = kk + lane
            for s in range(_NSLOT):
                plsc.store_scatter(bufs[s], [rv, kv], zeros)
            return 0
        lax.fori_loop(0, _CH * (_STEPS // _L), _zero, 0)

        for cp in xcopies:
            cp.wait()

        def _chunk(t, buf, posv, sem):
            jj = t // ch_per_j
            io = (t % ch_per_j) * _CH

            @pl.when(t >= _NSLOT)
            def _():
                pltpu.make_async_copy(
                    buf, out_hbm.at[jj, pl.ds(i_base + io, _CH)], sem).wait()
                for g in range(groups):
                    pv = posv[pl.ds(g * _L, _L)]
                    plsc.store_scatter(buf, [ivs[g], pv], zeros)

            for g in range(groups):
                xv = x_v[pl.ds(jj * n_per_w + io + g * _L, _L)]
                xs = (xv - _X_MIN) * ((_STEPS - 1) / (_X_MAX - _X_MIN))
                kv = jnp.clip(xs, 0.0, float(_STEPS - 1)).astype(jnp.int32)
                posv[pl.ds(g * _L, _L)] = kv
                plsc.store_scatter(buf, [ivs[g], kv], ones)

            pltpu.make_async_copy(
                buf, out_hbm.at[jj, pl.ds(i_base + io, _CH)], sem).start()

        def _step(t, _):
            for s in range(_NSLOT):
                _chunk(t * _NSLOT + s, bufs[s], poss[s], sems[s])
            return 0
        lax.fori_loop(0, n_chunks // _NSLOT, _step, 0)

        for s in range(_NSLOT):
            pltpu.make_async_copy(
                bufs[s], out_hbm.at[0, pl.ds(i_base, _CH)], sems[s]).wait()

    return k


def kernel(x, eye):
    del eye
    n, c = x.shape
    out_t = _make_sc_kernel(n, c)(x.T)
    return jnp.transpose(out_t, (1, 0, 2))


# speedup vs baseline: 5.9582x; 1.0301x over previous
import functools

import jax
import jax.numpy as jnp
from jax import lax
from jax.experimental import pallas as pl
from jax.experimental.pallas import tpu as pltpu
from jax.experimental.pallas import tpu_sc as plsc

_STEPS = 128
_X_MIN = 0.0
_X_MAX = 1.0
_L = 16
_CH = 64
_NSLOT = 8


def _make_sc_kernel(n: int, c: int):
    info = plsc.get_sparse_core_info()
    nc, ns = info.num_cores, info.num_subcores
    nw = nc * ns
    assert n % (nw * _CH) == 0
    n_per_w = n // nw
    ch_per_j = n_per_w // _CH
    n_chunks = c * ch_per_j
    groups = _CH // _L

    mesh = plsc.VectorSubcoreMesh(core_axis_name="c", subcore_axis_name="s")

    @functools.partial(
        pl.kernel,
        mesh=mesh,
        out_type=jax.ShapeDtypeStruct((c, n, _STEPS), jnp.float32),
        compiler_params=pltpu.CompilerParams(needs_layout_passes=False),
        scratch_types=(
            [pltpu.VMEM((c * n_per_w,), jnp.float32)]
            + [pltpu.VMEM((_CH, _STEPS), jnp.float32)] * _NSLOT
            + [pltpu.VMEM((_CH,), jnp.int32)] * _NSLOT
            + [pltpu.SemaphoreType.DMA] * (_NSLOT + 1)
        ),
    )
    def k(xt_hbm, out_hbm, *refs):
        x_v = refs[0]
        bufs = refs[1:1 + _NSLOT]
        poss = refs[1 + _NSLOT:1 + 2 * _NSLOT]
        sems = refs[1 + 2 * _NSLOT:1 + 3 * _NSLOT]
        xsem = refs[1 + 3 * _NSLOT]

        wid = lax.axis_index("s") * nc + lax.axis_index("c")
        i_base = wid * n_per_w

        xcopies = [
            pltpu.make_async_copy(xt_hbm.at[j, pl.ds(i_base, n_per_w)],
                                  x_v.at[pl.ds(j * n_per_w, n_per_w)], xsem)
            for j in range(c)
        ]
        for cp in xcopies:
            cp.start()

        lane = lax.iota(jnp.int32, _L)
        ones = jnp.full((_L,), 1.0, jnp.float32)
        zeros = jnp.zeros((_L,), jnp.float32)
        ivs = [g * _L + lane for g in range(groups)]

        def _zero(t, _):
            rr = t // (_STEPS // _L)
            kk = (t % (_STEPS // _L)) * _L
            rv = jnp.full((_L,), rr, jnp.int32)
            kv = kk + lane
            for s in range(_NSLOT):
                plsc.store_scatter(bufs[s], [rv, kv], zeros)
            return 0
        lax.fori_loop(0, _CH * (_STEPS // _L), _zero, 0)

        for cp in xcopies:
            cp.wait()

        def _chunk(t, buf, posv, sem):
            jj = t // ch_per_j
            io = (t % ch_per_j) * _CH

            @pl.when(t >= _NSLOT)
            def _():
                pltpu.make_async_copy(
                    buf, out_hbm.at[jj, pl.ds(i_base + io, _CH)], sem).wait()
                for g in range(groups):
                    pv = posv[pl.ds(g * _L, _L)]
                    plsc.store_scatter(buf, [ivs[g], pv], zeros)

            for g in range(groups):
                xv = x_v[pl.ds(jj * n_per_w + io + g * _L, _L)]
                xs = (xv - _X_MIN) * ((_STEPS - 1) / (_X_MAX - _X_MIN))
                kv = jnp.clip(xs, 0.0, float(_STEPS - 1)).astype(jnp.int32)
                posv[pl.ds(g * _L, _L)] = kv
                plsc.store_scatter(buf, [ivs[g], kv], ones)

            pltpu.make_async_copy(
                buf, out_hbm.at[jj, pl.ds(i_base + io, _CH)], sem).start()

        def _step(t, _):
            for s in range(_NSLOT):
                _chunk(t * _NSLOT + s, bufs[s], poss[s], sems[s])
            return 0
        lax.fori_loop(0, n_chunks // _NSLOT, _step, 0)

        for s in range(_NSLOT):
            pltpu.make_async_copy(
                bufs[s], out_hbm.at[0, pl.ds(i_base, _CH)], sems[s]).wait()

    return k


def kernel(x, eye):
    del eye
    n, c = x.shape
    out_t = _make_sc_kernel(n, c)(x.T)
    return jnp.transpose(out_t, (1, 0, 2))


# speedup vs baseline: 6.0314x; 1.0123x over previous
import functools

import jax
import jax.numpy as jnp
from jax import lax
from jax.experimental import pallas as pl
from jax.experimental.pallas import tpu as pltpu
from jax.experimental.pallas import tpu_sc as plsc

_STEPS = 128
_X_MIN = 0.0
_X_MAX = 1.0
_L = 16
_CH = 32
_NSLOT = 16


def _make_sc_kernel(n: int, c: int):
    info = plsc.get_sparse_core_info()
    nc, ns = info.num_cores, info.num_subcores
    nw = nc * ns
    assert n % (nw * _CH) == 0
    n_per_w = n // nw
    ch_per_j = n_per_w // _CH
    n_chunks = c * ch_per_j
    groups = _CH // _L

    mesh = plsc.VectorSubcoreMesh(core_axis_name="c", subcore_axis_name="s")

    @functools.partial(
        pl.kernel,
        mesh=mesh,
        out_type=jax.ShapeDtypeStruct((c, n, _STEPS), jnp.float32),
        compiler_params=pltpu.CompilerParams(needs_layout_passes=False),
        scratch_types=(
            [pltpu.VMEM((c * n_per_w,), jnp.float32)]
            + [pltpu.VMEM((_CH, _STEPS), jnp.float32)] * _NSLOT
            + [pltpu.VMEM((_CH,), jnp.int32)] * _NSLOT
            + [pltpu.SemaphoreType.DMA] * (_NSLOT + 1)
        ),
    )
    def k(xt_hbm, out_hbm, *refs):
        x_v = refs[0]
        bufs = refs[1:1 + _NSLOT]
        poss = refs[1 + _NSLOT:1 + 2 * _NSLOT]
        sems = refs[1 + 2 * _NSLOT:1 + 3 * _NSLOT]
        xsem = refs[1 + 3 * _NSLOT]

        wid = lax.axis_index("s") * nc + lax.axis_index("c")
        i_base = wid * n_per_w

        xcopies = [
            pltpu.make_async_copy(xt_hbm.at[j, pl.ds(i_base, n_per_w)],
                                  x_v.at[pl.ds(j * n_per_w, n_per_w)], xsem)
            for j in range(c)
        ]
        for cp in xcopies:
            cp.start()

        lane = lax.iota(jnp.int32, _L)
        ones = jnp.full((_L,), 1.0, jnp.float32)
        zeros = jnp.zeros((_L,), jnp.float32)
        ivs = [g * _L + lane for g in range(groups)]

        def _zero(t, _):
            rr = t // (_STEPS // _L)
            kk = (t % (_STEPS // _L)) * _L
            rv = jnp.full((_L,), rr, jnp.int32)
            kv = kk + lane
            for s in range(_NSLOT):
                plsc.store_scatter(bufs[s], [rv, kv], zeros)
            return 0
        lax.fori_loop(0, _CH * (_STEPS // _L), _zero, 0)

        for cp in xcopies:
            cp.wait()

        def _chunk(t, buf, posv, sem):
            jj = t // ch_per_j
            io = (t % ch_per_j) * _CH

            @pl.when(t >= _NSLOT)
            def _():
                pltpu.make_async_copy(
                    buf, out_hbm.at[jj, pl.ds(i_base + io, _CH)], sem).wait()
                for g in range(groups):
                    pv = posv[pl.ds(g * _L, _L)]
                    plsc.store_scatter(buf, [ivs[g], pv], zeros)

            for g in range(groups):
                xv = x_v[pl.ds(jj * n_per_w + io + g * _L, _L)]
                xs = (xv - _X_MIN) * ((_STEPS - 1) / (_X_MAX - _X_MIN))
                kv = jnp.clip(xs, 0.0, float(_STEPS - 1)).astype(jnp.int32)
                posv[pl.ds(g * _L, _L)] = kv
                plsc.store_scatter(buf, [ivs[g], kv], ones)

            pltpu.make_async_copy(
                buf, out_hbm.at[jj, pl.ds(i_base + io, _CH)], sem).start()

        def _step(t, _):
            for s in range(_NSLOT):
                _chunk(t * _NSLOT + s, bufs[s], poss[s], sems[s])
            return 0
        lax.fori_loop(0, n_chunks // _NSLOT, _step, 0)

        for s in range(_NSLOT):
            pltpu.make_async_copy(
                bufs[s], out_hbm.at[0, pl.ds(i_base, _CH)], sems[s]).wait()

    return k


def kernel(x, eye):
    del eye
    n, c = x.shape
    out_t = _make_sc_kernel(n, c)(x.T)
    return jnp.transpose(out_t, (1, 0, 2))
